# baseline (speedup 1.0000x reference)
import jax
import jax.numpy as jnp
from jax.experimental import pallas as pl
from jax.experimental.pallas import tpu as pltpu

_F32 = jnp.float32
_CP = pltpu.CompilerParams(vmem_limit_bytes=110 * 1024 * 1024)
_BF16 = jnp.bfloat16


def _prep_t(i, feat_refs, w_refs, t_ref):
    @pl.when(i == 0)
    def _():
        acc = None
        for f_ref, w_ref in zip(feat_refs, w_refs):
            p = jnp.dot(f_ref[...], w_ref[...].astype(_BF16),
                        preferred_element_type=_F32)
            acc = p if acc is None else acc + p
        t_ref[...] = acc.astype(_BF16)


def _pass1_kernel(adj_ref, x_ref, W0_ref, b0_ref, adjbf_ref, h0_ref, t_ref):
    _prep_t(pl.program_id(0), [x_ref], [W0_ref], t_ref)
    abf = adj_ref[...].astype(_BF16)
    adjbf_ref[...] = abf
    p0 = jnp.dot(abf, t_ref[...], preferred_element_type=_F32) + b0_ref[...]
    h0_ref[...] = jnp.tanh(p0).astype(_BF16)


def _pass2_kernel(adjbf_ref, x_ref, h0_ref, W1a_ref, W1b_ref, b1_ref,
                  h1_ref, t_ref):
    _prep_t(pl.program_id(0), [x_ref, h0_ref], [W1a_ref, W1b_ref], t_ref)
    p1 = (jnp.dot(adjbf_ref[...], t_ref[...], preferred_element_type=_F32)
          + b1_ref[...])
    h1_ref[...] = jnp.tanh(p1).astype(_BF16)


def _pass3_kernel(adjbf_ref, x_ref, h0_ref, h1_ref,
                  Woa_ref, Wob_ref, Woc_ref, bo_ref, out_ref, t_ref):
    _prep_t(pl.program_id(0), [x_ref, h0_ref, h1_ref],
            [Woa_ref, Wob_ref, Woc_ref], t_ref)
    logits = (jnp.dot(adjbf_ref[...], t_ref[...], preferred_element_type=_F32)
              + bo_ref[...])
    m = jnp.max(logits, axis=1, keepdims=True)
    shifted = logits - m
    out_ref[...] = shifted - jnp.log(
        jnp.sum(jnp.exp(shifted), axis=1, keepdims=True))


def kernel(x, adj, W0, b0, W1, b1, W_out, b_out):
    n, nf = x.shape
    nh = W0.shape[1]
    nc = W_out.shape[1]

    if n >= 1024:
        bm1, bm2 = 384, 1200
    else:
        bm1 = bm2 = min(256, n)
    grid1 = (pl.cdiv(n, bm1),)
    grid2 = (pl.cdiv(n, bm2),)

    x_bf = x.astype(_BF16)
    W1a, W1b = W1[:nf], W1[nf:]
    Woa, Wob, Woc = W_out[:nf], W_out[nf:nf + nh], W_out[nf + nh:]
    b0r = b0.reshape(1, nh)
    b1r = b1.reshape(1, nh)
    bor = b_out.reshape(1, nc)

    def row_spec(bm):
        return pl.BlockSpec((bm, n), lambda i: (i, 0))

    def full(a):
        return pl.BlockSpec(a.shape, lambda i: (0,) * a.ndim)

    def act_spec(bm, w):
        return pl.BlockSpec((bm, w), lambda i: (i, 0))

    adj_bf, h0 = pl.pallas_call(
        _pass1_kernel,
        grid=grid1,
        in_specs=[row_spec(bm1), full(x_bf), full(W0), full(b0r)],
        out_specs=[row_spec(bm1), act_spec(bm1, nh)],
        out_shape=[jax.ShapeDtypeStruct((n, n), _BF16),
                   jax.ShapeDtypeStruct((n, nh), _BF16)],
        scratch_shapes=[pltpu.VMEM((n, nh), _BF16)],
        compiler_params=_CP,
    )(adj, x_bf, W0, b0r)

    h1 = pl.pallas_call(
        _pass2_kernel,
        grid=grid2,
        in_specs=[row_spec(bm2), full(x_bf), full(h0), full(W1a), full(W1b),
                  full(b1r)],
        out_specs=act_spec(bm2, nh),
        out_shape=jax.ShapeDtypeStruct((n, nh), _BF16),
        scratch_shapes=[pltpu.VMEM((n, nh), _BF16)],
        compiler_params=_CP,
    )(adj_bf, x_bf, h0, W1a, W1b, b1r)

    logp = pl.pallas_call(
        _pass3_kernel,
        grid=grid2,
        in_specs=[row_spec(bm2), full(x_bf), full(h0), full(h1),
                  full(Woa), full(Wob), full(Woc), full(bor)],
        out_specs=act_spec(bm2, nc),
        out_shape=jax.ShapeDtypeStruct((n, nc), _F32),
        scratch_shapes=[pltpu.VMEM((n, nc), _BF16)],
        compiler_params=_CP,
    )(adj_bf, x_bf, h0, h1, Woa, Wob, Woc, bor)

    return logp


# speedup vs baseline: 1.0468x; 1.0468x over previous
import jax
import jax.numpy as jnp
from jax.experimental import pallas as pl
from jax.experimental.pallas import tpu as pltpu

_F32 = jnp.float32
_BF16 = jnp.bfloat16
_CP = pltpu.CompilerParams(vmem_limit_bytes=110 * 1024 * 1024)


def _matsum(feat_w_pairs):
    acc = None
    for f, w in feat_w_pairs:
        p = jnp.dot(f, w.astype(_BF16), preferred_element_type=_F32)
        acc = p if acc is None else acc + p
    return acc


def _pass1_kernel(adj_ref, x_ref, W0_ref, b0_ref, adjbf_ref, h0_ref, t_ref):
    @pl.when(pl.program_id(0) == 0)
    def _():
        t_ref[...] = _matsum([(x_ref[...], W0_ref[...])]).astype(_BF16)

    abf = adj_ref[...].astype(_BF16)
    adjbf_ref[...] = abf
    p0 = jnp.dot(abf, t_ref[...], preferred_element_type=_F32) + b0_ref[...]
    h0_ref[...] = jnp.tanh(p0).astype(_BF16)


def _make_pass23_kernel(nblk, bm, n):
    def _pass23_kernel(adjbf_ref, x_ref, h0_ref, W1a_ref, W1b_ref, b1_ref,
                       Woa_ref, Wob_ref, Woc_ref, bo_ref, out_ref,
                       t1_ref, t2_ref, h1_ref):
        i = pl.program_id(0)

        @pl.when(i == 0)
        def _():
            t1_ref[...] = _matsum([(x_ref[...], W1a_ref[...]),
                                   (h0_ref[...], W1b_ref[...])]).astype(_BF16)

        @pl.when(i < nblk)
        def _():
            p1 = (jnp.dot(adjbf_ref[...], t1_ref[...],
                          preferred_element_type=_F32) + b1_ref[...])
            h1_ref[pl.ds(i * bm, bm), :] = jnp.tanh(p1).astype(_BF16)

        @pl.when(i == nblk)
        def _():
            t2_ref[...] = _matsum([(x_ref[...], Woa_ref[...]),
                                   (h0_ref[...], Wob_ref[...]),
                                   (h1_ref[:n], Woc_ref[...])]).astype(_BF16)

        @pl.when(i >= nblk)
        def _():
            logits = (jnp.dot(adjbf_ref[...], t2_ref[...],
                              preferred_element_type=_F32) + bo_ref[...])
            m = jnp.max(logits, axis=1, keepdims=True)
            shifted = logits - m
            out_ref[...] = shifted - jnp.log(
                jnp.sum(jnp.exp(shifted), axis=1, keepdims=True))

    return _pass23_kernel


def kernel(x, adj, W0, b0, W1, b1, W_out, b_out):
    n, nf = x.shape
    nh = W0.shape[1]
    nc = W_out.shape[1]

    if n >= 1024:
        bm1, bm2 = 384, 1024
    else:
        bm1 = bm2 = min(256, n)
    k1 = pl.cdiv(n, bm1)
    k2 = pl.cdiv(n, bm2)

    x_bf = x.astype(_BF16)
    W1a, W1b = W1[:nf], W1[nf:]
    Woa, Wob, Woc = W_out[:nf], W_out[nf:nf + nh], W_out[nf + nh:]
    b0r = b0.reshape(1, nh)
    b1r = b1.reshape(1, nh)
    bor = b_out.reshape(1, nc)

    def full(a):
        return pl.BlockSpec(a.shape, lambda i: (0,) * a.ndim)

    adj_bf, h0 = pl.pallas_call(
        _pass1_kernel,
        grid=(k1,),
        in_specs=[pl.BlockSpec((bm1, n), lambda i: (i, 0)), full(x_bf),
                  full(W0), full(b0r)],
        out_specs=[pl.BlockSpec((bm1, n), lambda i: (i, 0)),
                   pl.BlockSpec((bm1, nh), lambda i: (i, 0))],
        out_shape=[jax.ShapeDtypeStruct((n, n), _BF16),
                   jax.ShapeDtypeStruct((n, nh), _BF16)],
        scratch_shapes=[pltpu.VMEM((n, nh), _BF16)],
        compiler_params=_CP,
    )(adj, x_bf, W0, b0r)

    logp = pl.pallas_call(
        _make_pass23_kernel(k2, bm2, n),
        grid=(2 * k2,),
        in_specs=[pl.BlockSpec((bm2, n), lambda i: (i % k2, 0)), full(x_bf),
                  full(h0), full(W1a), full(W1b), full(b1r),
                  full(Woa), full(Wob), full(Woc), full(bor)],
        out_specs=pl.BlockSpec((bm2, nc),
                               lambda i: (jnp.maximum(i - k2, 0), 0)),
        out_shape=jax.ShapeDtypeStruct((n, nc), _F32),
        scratch_shapes=[pltpu.VMEM((n, nh), _BF16),
                        pltpu.VMEM((n, nc), _BF16),
                        pltpu.VMEM((k2 * bm2, nh), _BF16)],
        compiler_params=_CP,
    )(adj_bf, x_bf, h0, W1a, W1b, b1r, Woa, Wob, Woc, bor)

    return logp


# speedup vs baseline: 1.0514x; 1.0044x over previous
import jax
import jax.numpy as jnp
from jax.experimental import pallas as pl
from jax.experimental.pallas import tpu as pltpu

_F32 = jnp.float32
_BF16 = jnp.bfloat16
_CP = pltpu.CompilerParams(vmem_limit_bytes=110 * 1024 * 1024)


def _matsum(feat_w_pairs):
    acc = None
    for f, w in feat_w_pairs:
        p = jnp.dot(f, w.astype(_BF16), preferred_element_type=_F32)
        acc = p if acc is None else acc + p
    return acc


def _pass1_kernel(adj_ref, x_ref, W0_ref, b0_ref, adjbf_ref, h0_ref, t_ref):
    @pl.when(pl.program_id(0) == 0)
    def _():
        t_ref[...] = _matsum([(x_ref[...], W0_ref[...])]).astype(_BF16)

    abf = adj_ref[...].astype(_BF16)
    adjbf_ref[...] = abf
    p0 = jnp.dot(abf, t_ref[...], preferred_element_type=_F32) + b0_ref[...]
    h0_ref[...] = jnp.tanh(p0).astype(_BF16)


def _make_pass23_kernel(nblk, bm, n):
    def _pass23_kernel(adjbf_ref, x_ref, h0_ref, W1a_ref, W1b_ref, b1_ref,
                       Woa_ref, Wob_ref, Woc_ref, bo_ref, out_ref,
                       t1_ref, t2_ref, h1_ref):
        i = pl.program_id(0)

        @pl.when(i == 0)
        def _():
            t1_ref[...] = _matsum([(x_ref[...], W1a_ref[...]),
                                   (h0_ref[...], W1b_ref[...])]).astype(_BF16)

        @pl.when(i < nblk)
        def _():
            p1 = (jnp.dot(adjbf_ref[...], t1_ref[...],
                          preferred_element_type=_F32) + b1_ref[...])
            h1_ref[pl.ds(i * bm, bm), :] = jnp.tanh(p1).astype(_BF16)

        @pl.when(i == nblk)
        def _():
            t2_ref[...] = _matsum([(x_ref[...], Woa_ref[...]),
                                   (h0_ref[...], Wob_ref[...]),
                                   (h1_ref[:n], Woc_ref[...])]).astype(_BF16)

        @pl.when(i >= nblk)
        def _():
            logits = (jnp.dot(adjbf_ref[...], t2_ref[...],
                              preferred_element_type=_F32) + bo_ref[...])
            m = jnp.max(logits, axis=1, keepdims=True)
            shifted = logits - m
            out_ref[...] = shifted - jnp.log(
                jnp.sum(jnp.exp(shifted), axis=1, keepdims=True))

    return _pass23_kernel


def kernel(x, adj, W0, b0, W1, b1, W_out, b_out):
    n, nf = x.shape
    nh = W0.shape[1]
    nc = W_out.shape[1]

    if n % 2000 == 0:
        bm1, bm2 = 400, 1000
    elif n >= 1024:
        bm1, bm2 = 384, 1024
    else:
        bm1 = bm2 = min(256, n)
    k1 = pl.cdiv(n, bm1)
    k2 = pl.cdiv(n, bm2)

    x_bf = x.astype(_BF16)
    W1a, W1b = W1[:nf], W1[nf:]
    Woa, Wob, Woc = W_out[:nf], W_out[nf:nf + nh], W_out[nf + nh:]
    b0r = b0.reshape(1, nh)
    b1r = b1.reshape(1, nh)
    bor = b_out.reshape(1, nc)

    def full(a):
        return pl.BlockSpec(a.shape, lambda i: (0,) * a.ndim)

    adj_bf, h0 = pl.pallas_call(
        _pass1_kernel,
        grid=(k1,),
        in_specs=[pl.BlockSpec((bm1, n), lambda i: (i, 0)), full(x_bf),
                  full(W0), full(b0r)],
        out_specs=[pl.BlockSpec((bm1, n), lambda i: (i, 0)),
                   pl.BlockSpec((bm1, nh), lambda i: (i, 0))],
        out_shape=[jax.ShapeDtypeStruct((n, n), _BF16),
                   jax.ShapeDtypeStruct((n, nh), _BF16)],
        scratch_shapes=[pltpu.VMEM((n, nh), _BF16)],
        compiler_params=_CP,
    )(adj, x_bf, W0, b0r)

    logp = pl.pallas_call(
        _make_pass23_kernel(k2, bm2, n),
        grid=(2 * k2,),
        in_specs=[pl.BlockSpec((bm2, n), lambda i: (i % k2, 0)), full(x_bf),
                  full(h0), full(W1a), full(W1b), full(b1r),
                  full(Woa), full(Wob), full(Woc), full(bor)],
        out_specs=pl.BlockSpec((bm2, nc),
                               lambda i: (jnp.maximum(i - k2, 0), 0)),
        out_shape=jax.ShapeDtypeStruct((n, nc), _F32),
        scratch_shapes=[pltpu.VMEM((n, nh), _BF16),
                        pltpu.VMEM((n, nc), _BF16),
                        pltpu.VMEM((k2 * bm2, nh), _BF16)],
        compiler_params=_CP,
    )(adj_bf, x_bf, h0, W1a, W1b, b1r, Woa, Wob, Woc, bor)

    return logp


# speedup vs baseline: 1.0728x; 1.0203x over previous
import jax
import jax.numpy as jnp
from jax.experimental import pallas as pl
from jax.experimental.pallas import tpu as pltpu

_F32 = jnp.float32
_BF16 = jnp.bfloat16
_CP = pltpu.CompilerParams(vmem_limit_bytes=110 * 1024 * 1024)


def _matsum(feat_w_pairs):
    acc = None
    for f, w in feat_w_pairs:
        p = jnp.dot(f.astype(_BF16), w.astype(_BF16),
                    preferred_element_type=_F32)
        acc = p if acc is None else acc + p
    return acc


def _pass1_kernel(adj_ref, x_ref, W0_ref, b0_ref, adjbf_ref, xbf_ref,
                  h0_ref, t_ref):
    @pl.when(pl.program_id(0) == 0)
    def _():
        xbf = x_ref[...].astype(_BF16)
        xbf_ref[...] = xbf
        t_ref[...] = _matsum([(xbf, W0_ref[...])]).astype(_BF16)

    abf = adj_ref[...].astype(_BF16)
    adjbf_ref[...] = abf
    p0 = jnp.dot(abf, t_ref[...], preferred_element_type=_F32) + b0_ref[...]
    h0_ref[...] = jnp.tanh(p0).astype(_BF16)


def _make_pass23_kernel(nblk, bm, n, nf, nh):
    def _pass23_kernel(adjbf_ref, x_ref, h0_ref, W1_ref, b1_ref,
                       Wo_ref, bo_ref, out_ref, t1_ref, t2_ref, h1_ref):
        i = pl.program_id(0)

        @pl.when(i == 0)
        def _():
            t1_ref[...] = _matsum([(x_ref[...], W1_ref[:nf]),
                                   (h0_ref[...], W1_ref[nf:])]).astype(_BF16)

        @pl.when(i < nblk)
        def _():
            p1 = (jnp.dot(adjbf_ref[...], t1_ref[...],
                          preferred_element_type=_F32) + b1_ref[...])
            h1_ref[pl.ds(i * bm, bm), :] = jnp.tanh(p1).astype(_BF16)

        @pl.when(i == nblk)
        def _():
            t2_ref[...] = _matsum([(x_ref[...], Wo_ref[:nf]),
                                   (h0_ref[...], Wo_ref[nf:nf + nh]),
                                   (h1_ref[:n], Wo_ref[nf + nh:])]).astype(_BF16)

        @pl.when(i >= nblk)
        def _():
            logits = (jnp.dot(adjbf_ref[...], t2_ref[...],
                              preferred_element_type=_F32) + bo_ref[...])
            m = jnp.max(logits, axis=1, keepdims=True)
            shifted = logits - m
            out_ref[...] = shifted - jnp.log(
                jnp.sum(jnp.exp(shifted), axis=1, keepdims=True))

    return _pass23_kernel


def kernel(x, adj, W0, b0, W1, b1, W_out, b_out):
    n, nf = x.shape
    nh = W0.shape[1]
    nc = W_out.shape[1]

    if n % 2000 == 0:
        bm1, bm2 = 400, 1000
    elif n >= 1024:
        bm1, bm2 = 384, 1024
    else:
        bm1 = bm2 = min(256, n)
    k1 = pl.cdiv(n, bm1)
    k2 = pl.cdiv(n, bm2)

    b0r = b0.reshape(1, nh)
    b1r = b1.reshape(1, nh)
    bor = b_out.reshape(1, nc)

    def full(a):
        return pl.BlockSpec(a.shape, lambda i: (0,) * a.ndim)

    adj_bf, x_bf, h0 = pl.pallas_call(
        _pass1_kernel,
        grid=(k1,),
        in_specs=[pl.BlockSpec((bm1, n), lambda i: (i, 0)), full(x),
                  full(W0), full(b0r)],
        out_specs=[pl.BlockSpec((bm1, n), lambda i: (i, 0)),
                   pl.BlockSpec((n, nf), lambda i: (0, 0)),
                   pl.BlockSpec((bm1, nh), lambda i: (i, 0))],
        out_shape=[jax.ShapeDtypeStruct((n, n), _BF16),
                   jax.ShapeDtypeStruct((n, nf), _BF16),
                   jax.ShapeDtypeStruct((n, nh), _BF16)],
        scratch_shapes=[pltpu.VMEM((n, nh), _BF16)],
        compiler_params=_CP,
    )(adj, x, W0, b0r)

    logp = pl.pallas_call(
        _make_pass23_kernel(k2, bm2, n, nf, nh),
        grid=(2 * k2,),
        in_specs=[pl.BlockSpec((bm2, n), lambda i: (i % k2, 0)), full(x_bf),
                  full(h0), full(W1), full(b1r), full(W_out), full(bor)],
        out_specs=pl.BlockSpec((bm2, nc),
                               lambda i: (jnp.maximum(i - k2, 0), 0)),
        out_shape=jax.ShapeDtypeStruct((n, nc), _F32),
        scratch_shapes=[pltpu.VMEM((n, nh), _BF16),
                        pltpu.VMEM((n, nc), _BF16),
                        pltpu.VMEM((k2 * bm2, nh), _BF16)],
        compiler_params=_CP,
    )(adj_bf, x_bf, h0, W1, b1r, W_out, bor)

    return logp
